# baseline (speedup 1.0000x reference)
import functools

import jax
import jax.numpy as jnp
from jax import lax
from jax.experimental import pallas as pl
from jax.experimental.pallas import tpu as pltpu
from jax.experimental.pallas import tpu_sc as plsc

VOCAB = 1000
B, T = 4096, 50
BT = B * T
TP = 64

NC, NS, L = 2, 16, 16
NW = NC * NS
BPW = B // NW
NG = 4


def _lse_body(table_ref, out_ref):
    t = table_ref[...]
    m = jnp.max(t, axis=1, keepdims=True)
    out_ref[...] = m + jnp.log(jnp.sum(jnp.exp(t - m), axis=1, keepdims=True))


def _row_lse(table):
    return pl.pallas_call(
        _lse_body,
        out_shape=jax.ShapeDtypeStruct((VOCAB, 1), jnp.float32),
    )(table)


_MESH = plsc.VectorSubcoreMesh(core_axis_name="c", subcore_axis_name="s")


@functools.partial(
    pl.kernel,
    out_type=(
        jax.ShapeDtypeStruct((B, T, VOCAB), jnp.float32),
        jax.ShapeDtypeStruct((NW, L), jnp.float32),
    ),
    mesh=_MESH,
    compiler_params=pltpu.CompilerParams(
        needs_layout_passes=False, use_tc_tiling_on_sc=False),
    scratch_types=[
        pltpu.VMEM((BPW, T), jnp.int32),
        pltpu.VMEM((BPW, TP), jnp.int32),
        pltpu.VMEM((BPW, TP), jnp.int32),
        pltpu.VMEM((1, VOCAB), jnp.float32),
        pltpu.VMEM((1, T, VOCAB), jnp.float32),
        pltpu.VMEM((L,), jnp.float32),
        pltpu.SemaphoreType.DMA,
    ],
)
def _sc_gather_loss(table_hbm, idx_hbm, idxp_hbm, tgtp_hbm, lse_hbm,
                    logits_hbm, part_hbm,
                    idx_v, idxp_v, tgtp_v, lse_v, rows_v, acc_v, sem):
    wid = lax.axis_index("s") * NC + lax.axis_index("c")
    b0 = wid * BPW
    pltpu.sync_copy(idx_hbm.at[pl.ds(b0, BPW)], idx_v)
    pltpu.sync_copy(idxp_hbm.at[pl.ds(b0, BPW)], idxp_v)
    pltpu.sync_copy(tgtp_hbm.at[pl.ds(b0, BPW)], tgtp_v)
    pltpu.sync_copy(lse_hbm, lse_v)

    lane = lax.iota(jnp.int32, L)
    zero = jnp.zeros((L,), jnp.int32)

    def b_body(k, acc):
        pltpu.async_copy(table_hbm.at[idx_v.at[k]], rows_v.at[0], sem).wait()
        pltpu.sync_copy(rows_v, logits_hbm.at[pl.ds(b0 + k, 1)])
        for j in range(NG):
            t_id = lane + j * L
            live = t_id < T
            iv = idxp_v[k, pl.ds(j * L, L)]
            tv = tgtp_v[k, pl.ds(j * L, L)]
            lsev = plsc.load_gather(lse_v, [zero, iv], mask=live)
            picks = plsc.load_gather(rows_v, [zero, t_id, tv], mask=live)
            acc = acc + jnp.where(live, lsev - picks, 0.0)
        return acc

    acc = lax.fori_loop(0, BPW, b_body, jnp.zeros((L,), jnp.float32))
    acc_v[...] = acc
    pltpu.sync_copy(acc_v, part_hbm.at[wid])


def kernel(idx, targets, table):
    lse = _row_lse(table).reshape(1, VOCAB)
    idxp = jnp.pad(idx, ((0, 0), (0, TP - T)))
    tgtp = jnp.pad(targets, ((0, 0), (0, TP - T)))
    logits, parts = _sc_gather_loss(table, idx, idxp, tgtp, lse)
    loss = jnp.sum(parts) / BT
    return (logits, loss)


# speedup vs baseline: 1.2118x; 1.2118x over previous
import functools

import jax
import jax.numpy as jnp
from jax import lax
from jax.experimental import pallas as pl
from jax.experimental.pallas import tpu as pltpu
from jax.experimental.pallas import tpu_sc as plsc

VOCAB = 1000
B, T = 4096, 50
BT = B * T

NC, NS, L = 2, 16, 16
NW = NC * NS
NCR = VOCAB // 8
KPW = 4
TILE = 8 * B
NG = B // L


def _lse_body(table_ref, out_ref):
    t = table_ref[...]
    m = jnp.max(t, axis=1, keepdims=True)
    out_ref[...] = m + jnp.log(jnp.sum(jnp.exp(t - m), axis=1, keepdims=True))


def _row_lse(table):
    return pl.pallas_call(
        _lse_body,
        out_shape=jax.ShapeDtypeStruct((VOCAB, 1), jnp.float32),
    )(table)


_MESH = plsc.VectorSubcoreMesh(core_axis_name="c", subcore_axis_name="s")


@functools.partial(
    pl.kernel,
    out_type=(
        jax.ShapeDtypeStruct((T, NCR, TILE), jnp.float32),
        jax.ShapeDtypeStruct((NW, L), jnp.float32),
    ),
    mesh=_MESH,
    compiler_params=pltpu.CompilerParams(
        needs_layout_passes=False, use_tc_tiling_on_sc=False),
    scratch_types=[
        pltpu.VMEM((1, KPW * 8 * VOCAB), jnp.float32),
        pltpu.VMEM((B,), jnp.int32),
        pltpu.VMEM((B,), jnp.int32),
        pltpu.VMEM((1, VOCAB), jnp.float32),
        pltpu.VMEM((2, TILE), jnp.float32),
        pltpu.VMEM((L,), jnp.float32),
        pltpu.SemaphoreType.DMA,
    ],
)
def _sc_gather_loss(tableT_hbm, idxT_hbm, tgtT_hbm, lse_hbm,
                    out_hbm, part_hbm,
                    tab_v, idx_v, tgt_v, lse_v, obuf_v, acc_v, sem):
    wid = lax.axis_index("s") * NC + lax.axis_index("c")
    cr0 = wid * KPW

    for k in range(KPW):
        crk = jnp.minimum(cr0 + k, NCR - 1)
        pltpu.sync_copy(tableT_hbm.at[pl.ds(crk * 8 * VOCAB, 8 * VOCAB)],
                        tab_v.at[0, pl.ds(k * 8 * VOCAB, 8 * VOCAB)])
    pltpu.sync_copy(lse_hbm, lse_v)

    zero = jnp.zeros((L,), jnp.int32)

    def t_body(t, acc):
        pltpu.sync_copy(idxT_hbm.at[pl.ds(t * B, B)], idx_v)
        pltpu.sync_copy(tgtT_hbm.at[pl.ds(t * B, B)], tgt_v)

        for k in range(KPW):
            crk = jnp.minimum(cr0 + k, NCR - 1)
            par = (t * KPW + k) % 2
            drain = pltpu.make_async_copy(
                obuf_v.at[par], out_hbm.at[t, crk], sem)
            if k >= 2:
                drain.wait()
            else:
                @pl.when(t >= 1)
                def _():
                    drain.wait()

            def g_body(g, carry):
                iv = idx_v[pl.ds(g * L, L)]
                off = (g >> 3) * 1024 + (g & 7) * L
                for s in range(8):
                    vals = plsc.load_gather(
                        tab_v, [zero, iv + (k * 8 * VOCAB + s * VOCAB)])
                    obuf_v[par, pl.ds(off + s * 128, L)] = vals
                return carry

            lax.fori_loop(0, NG, g_body, 0, unroll=False)
            pltpu.async_copy(obuf_v.at[par], out_hbm.at[t, crk], sem)

        def loss_body(g, a):
            iv = idx_v[pl.ds(g * L, L)]
            tv = tgt_v[pl.ds(g * L, L)]
            rel = (tv >> 3) - cr0
            m = (rel >= 0) & (rel < KPW)
            relc = jnp.clip(rel, 0, KPW - 1)
            addr = relc * (8 * VOCAB) + (tv & 7) * VOCAB + iv
            picks = plsc.load_gather(tab_v, [zero, addr], mask=m)
            lsev = plsc.load_gather(lse_v, [zero, iv], mask=m)
            return a + jnp.where(m, lsev - picks, 0.0)

        return lax.fori_loop(0, NG, loss_body, acc)

    acc = lax.fori_loop(0, T, t_body, jnp.zeros((L,), jnp.float32))

    for j in range(2):
        pltpu.make_async_copy(obuf_v.at[j], out_hbm.at[0, 0], sem).wait()

    acc_v[...] = acc
    pltpu.sync_copy(acc_v, part_hbm.at[wid])


def kernel(idx, targets, table):
    lse = _row_lse(table).reshape(1, VOCAB)
    tableT_flat = table.T.reshape(VOCAB * VOCAB)
    idxT_flat = idx.T.reshape(BT)
    tgtT_flat = targets.T.reshape(BT)
    out5, parts = _sc_gather_loss(tableT_flat, idxT_flat, tgtT_flat, lse)
    logits = (out5.reshape(T, NCR, B // 128, 8, 128)
              .transpose(2, 4, 0, 1, 3)
              .reshape(B, T, VOCAB))
    loss = jnp.sum(parts) / BT
    return (logits, loss)


# speedup vs baseline: 4.6791x; 3.8611x over previous
import functools

import jax
import jax.numpy as jnp
from jax import lax
from jax.experimental import pallas as pl
from jax.experimental.pallas import tpu as pltpu
from jax.experimental.pallas import tpu_sc as plsc

VOCAB = 1000
B, T = 4096, 50
BT = B * T

NC, NS, L = 2, 16, 16
NW = NC * NS
NCR = VOCAB // 8
KPW = 4
TILE = 8 * B
NG = B // L


def _lse_body(table_ref, out_ref):
    t = table_ref[...]
    m = jnp.max(t, axis=1, keepdims=True)
    out_ref[...] = m + jnp.log(jnp.sum(jnp.exp(t - m), axis=1, keepdims=True))


def _row_lse(table):
    return pl.pallas_call(
        _lse_body,
        out_shape=jax.ShapeDtypeStruct((VOCAB, 1), jnp.float32),
    )(table)


_MESH = plsc.VectorSubcoreMesh(core_axis_name="c", subcore_axis_name="s")


@functools.partial(
    pl.kernel,
    out_type=(
        jax.ShapeDtypeStruct((T, NCR, TILE), jnp.float32),
        jax.ShapeDtypeStruct((NW, L), jnp.float32),
    ),
    mesh=_MESH,
    compiler_params=pltpu.CompilerParams(
        needs_layout_passes=False, use_tc_tiling_on_sc=False),
    scratch_types=[
        pltpu.VMEM((1, KPW * 8 * VOCAB), jnp.float32),
        pltpu.VMEM((B,), jnp.int32),
        pltpu.VMEM((B,), jnp.int32),
        pltpu.VMEM((1, VOCAB), jnp.float32),
        pltpu.VMEM((2, TILE), jnp.float32),
        pltpu.VMEM((L,), jnp.float32),
        pltpu.SemaphoreType.DMA,
    ],
)
def _sc_gather_loss(tableT_hbm, idxT_hbm, tgtT_hbm, lse_hbm,
                    out_hbm, part_hbm,
                    tab_v, idx_v, tgt_v, lse_v, obuf_v, acc_v, sem):
    wid = lax.axis_index("s") * NC + lax.axis_index("c")
    cr0 = wid * KPW

    for k in range(KPW):
        crk = jnp.minimum(cr0 + k, NCR - 1)
        pltpu.sync_copy(tableT_hbm.at[pl.ds(crk * 8 * VOCAB, 8 * VOCAB)],
                        tab_v.at[0, pl.ds(k * 8 * VOCAB, 8 * VOCAB)])
    pltpu.sync_copy(lse_hbm, lse_v)

    zero = jnp.zeros((L,), jnp.int32)

    def t_body(t, acc):
        pltpu.sync_copy(idxT_hbm.at[pl.ds(t * B, B)], idx_v)
        pltpu.sync_copy(tgtT_hbm.at[pl.ds(t * B, B)], tgt_v)

        for k in range(KPW):
            crk = jnp.minimum(cr0 + k, NCR - 1)
            par = (t * KPW + k) % 2
            drain = pltpu.make_async_copy(
                obuf_v.at[par], out_hbm.at[t, crk], sem)
            if k >= 2:
                drain.wait()
            else:
                @pl.when(t >= 1)
                def _():
                    drain.wait()

            @plsc.parallel_loop(0, NG, 1, unroll=4)
            def _(g):
                iv = idx_v[pl.ds(g * L, L)]
                off = (g >> 3) * 1024 + (g & 7) * L
                for s in range(8):
                    vals = plsc.load_gather(
                        tab_v, [zero, iv + (k * 8 * VOCAB + s * VOCAB)])
                    obuf_v[par, pl.ds(off + s * 128, L)] = vals

            pltpu.async_copy(obuf_v.at[par], out_hbm.at[t, crk], sem)

        @plsc.parallel_loop(0, NG, 1, unroll=4, carry=acc)
        def loss_acc(g, a):
            iv = idx_v[pl.ds(g * L, L)]
            tv = tgt_v[pl.ds(g * L, L)]
            rel = (tv >> 3) - cr0
            m = (rel >= 0) & (rel < KPW)
            relc = jnp.clip(rel, 0, KPW - 1)
            addr = relc * (8 * VOCAB) + (tv & 7) * VOCAB + iv
            picks = plsc.load_gather(tab_v, [zero, addr], mask=m)
            lsev = plsc.load_gather(lse_v, [zero, iv], mask=m)
            return a + jnp.where(m, lsev - picks, 0.0)

        return loss_acc

    acc = lax.fori_loop(0, T, t_body, jnp.zeros((L,), jnp.float32))

    for j in range(2):
        pltpu.make_async_copy(obuf_v.at[j], out_hbm.at[0, 0], sem).wait()

    acc_v[...] = acc
    pltpu.sync_copy(acc_v, part_hbm.at[wid])


def kernel(idx, targets, table):
    lse = _row_lse(table).reshape(1, VOCAB)
    tableT_flat = table.T.reshape(VOCAB * VOCAB)
    idxT_flat = idx.T.reshape(BT)
    tgtT_flat = targets.T.reshape(BT)
    out5, parts = _sc_gather_loss(tableT_flat, idxT_flat, tgtT_flat, lse)
    logits = (out5.reshape(T, NCR, B // 128, 8, 128)
              .transpose(2, 4, 0, 1, 3)
              .reshape(B, T, VOCAB))
    loss = jnp.sum(parts) / BT
    return (logits, loss)


# speedup vs baseline: 4.7133x; 1.0073x over previous
import functools

import jax
import jax.numpy as jnp
from jax import lax
from jax.experimental import pallas as pl
from jax.experimental.pallas import tpu as pltpu
from jax.experimental.pallas import tpu_sc as plsc

VOCAB = 1000
B, T = 4096, 50
BT = B * T

NC, NS, L = 2, 16, 16
NW = NC * NS
NCR = VOCAB // 8
KPW = 4
TILE = 8 * B
NG = B // L


def _lse_body(table_ref, out_ref):
    t = table_ref[...]
    m = jnp.max(t, axis=1, keepdims=True)
    out_ref[...] = m + jnp.log(jnp.sum(jnp.exp(t - m), axis=1, keepdims=True))


def _row_lse(table):
    return pl.pallas_call(
        _lse_body,
        out_shape=jax.ShapeDtypeStruct((VOCAB, 1), jnp.float32),
    )(table)


_MESH = plsc.VectorSubcoreMesh(core_axis_name="c", subcore_axis_name="s")


@functools.partial(
    pl.kernel,
    out_type=(
        jax.ShapeDtypeStruct((T, NCR, TILE), jnp.float32),
        jax.ShapeDtypeStruct((NW, L), jnp.float32),
    ),
    mesh=_MESH,
    compiler_params=pltpu.CompilerParams(
        needs_layout_passes=False, use_tc_tiling_on_sc=False),
    scratch_types=[
        pltpu.VMEM((1, KPW * 8 * VOCAB), jnp.float32),
        pltpu.VMEM((B,), jnp.int32),
        pltpu.VMEM((B,), jnp.int32),
        pltpu.VMEM((1, VOCAB), jnp.float32),
        pltpu.VMEM((2, TILE), jnp.float32),
        pltpu.VMEM((L,), jnp.float32),
        pltpu.SemaphoreType.DMA,
    ],
)
def _sc_gather_loss(tableT_hbm, idxT_hbm, tgtT_hbm, lse_hbm,
                    out_hbm, part_hbm,
                    tab_v, idx_v, tgt_v, lse_v, obuf_v, acc_v, sem):
    wid = lax.axis_index("s") * NC + lax.axis_index("c")
    cr0 = wid * KPW

    for k in range(KPW):
        crk = jnp.minimum(cr0 + k, NCR - 1)
        pltpu.sync_copy(tableT_hbm.at[pl.ds(crk * 8 * VOCAB, 8 * VOCAB)],
                        tab_v.at[0, pl.ds(k * 8 * VOCAB, 8 * VOCAB)])
    pltpu.sync_copy(lse_hbm, lse_v)

    zero = jnp.zeros((L,), jnp.int32)

    def t_body(t, acc):
        pltpu.sync_copy(idxT_hbm.at[pl.ds(t * B, B)], idx_v)
        pltpu.sync_copy(tgtT_hbm.at[pl.ds(t * B, B)], tgt_v)

        for k in range(KPW):
            crk = jnp.minimum(cr0 + k, NCR - 1)
            par = (t * KPW + k) % 2
            drain = pltpu.make_async_copy(
                obuf_v.at[par], out_hbm.at[t, crk], sem)
            if k >= 2:
                drain.wait()
            else:
                @pl.when(t >= 1)
                def _():
                    drain.wait()

            @plsc.parallel_loop(0, NG, 1, unroll=8)
            def _(g):
                iv = idx_v[pl.ds(g * L, L)]
                off = (g >> 3) * 1024 + (g & 7) * L
                for s in range(8):
                    vals = plsc.load_gather(
                        tab_v, [zero, iv + (k * 8 * VOCAB + s * VOCAB)])
                    obuf_v[par, pl.ds(off + s * 128, L)] = vals

            pltpu.async_copy(obuf_v.at[par], out_hbm.at[t, crk], sem)

        @plsc.parallel_loop(0, NG, 1, unroll=4, carry=acc)
        def loss_acc(g, a):
            iv = idx_v[pl.ds(g * L, L)]
            tv = tgt_v[pl.ds(g * L, L)]
            rel = (tv >> 3) - cr0
            m = (rel >= 0) & (rel < KPW)
            relc = jnp.clip(rel, 0, KPW - 1)
            addr = relc * (8 * VOCAB) + (tv & 7) * VOCAB + iv
            picks = plsc.load_gather(tab_v, [zero, addr], mask=m)
            lsev = plsc.load_gather(lse_v, [zero, iv], mask=m)
            return a + jnp.where(m, lsev - picks, 0.0)

        return loss_acc

    acc = lax.fori_loop(0, T, t_body, jnp.zeros((L,), jnp.float32))

    for j in range(2):
        pltpu.make_async_copy(obuf_v.at[j], out_hbm.at[0, 0], sem).wait()

    acc_v[...] = acc
    pltpu.sync_copy(acc_v, part_hbm.at[wid])


def kernel(idx, targets, table):
    lse = _row_lse(table).reshape(1, VOCAB)
    tableT_flat = table.T.reshape(VOCAB * VOCAB)
    idxT_flat = idx.T.reshape(BT)
    tgtT_flat = targets.T.reshape(BT)
    out5, parts = _sc_gather_loss(tableT_flat, idxT_flat, tgtT_flat, lse)
    logits = (out5.reshape(T, NCR, B // 128, 8, 128)
              .transpose(2, 4, 0, 1, 3)
              .reshape(B, T, VOCAB))
    loss = jnp.sum(parts) / BT
    return (logits, loss)


# speedup vs baseline: 5.9671x; 1.2660x over previous
import functools

import jax
import jax.numpy as jnp
from jax import lax
from jax.experimental import pallas as pl
from jax.experimental.pallas import tpu as pltpu
from jax.experimental.pallas import tpu_sc as plsc

VOCAB = 1000
B, T = 4096, 50
BT = B * T

NC, NS, L = 2, 16, 16
NW = NC * NS
NCR = VOCAB // 8
KPW = 4
TILE = 8 * B
NG = B // L


def _lse_body(table_ref, out_ref):
    t = table_ref[...]
    m = jnp.max(t, axis=1, keepdims=True)
    out_ref[...] = m + jnp.log(jnp.sum(jnp.exp(t - m), axis=1, keepdims=True))


def _row_lse(table):
    return pl.pallas_call(
        _lse_body,
        out_shape=jax.ShapeDtypeStruct((VOCAB, 1), jnp.float32),
    )(table)


_MESH = plsc.VectorSubcoreMesh(core_axis_name="c", subcore_axis_name="s")


@functools.partial(
    pl.kernel,
    out_type=(
        jax.ShapeDtypeStruct((T, NCR, TILE), jnp.float32),
        jax.ShapeDtypeStruct((NW, L), jnp.float32),
    ),
    mesh=_MESH,
    compiler_params=pltpu.CompilerParams(
        needs_layout_passes=False, use_tc_tiling_on_sc=False),
    scratch_types=[
        pltpu.VMEM((1, KPW * 8 * VOCAB), jnp.float32),
        pltpu.VMEM((2, B), jnp.int32),
        pltpu.VMEM((2, B), jnp.int32),
        pltpu.VMEM((1, VOCAB), jnp.float32),
        pltpu.VMEM((2, TILE), jnp.float32),
        pltpu.VMEM((L,), jnp.float32),
        pltpu.SemaphoreType.DMA,
        pltpu.SemaphoreType.DMA,
    ],
)
def _sc_gather_loss(tableT_hbm, idxT_hbm, tgtT_hbm, lse_hbm,
                    out_hbm, part_hbm,
                    tab_v, idx_v, tgt_v, lse_v, obuf_v, acc_v, sem, sem2):
    wid = lax.axis_index("s") * NC + lax.axis_index("c")
    cr0 = wid * KPW

    for k in range(KPW):
        crk = jnp.minimum(cr0 + k, NCR - 1)
        pltpu.sync_copy(tableT_hbm.at[pl.ds(crk * 8 * VOCAB, 8 * VOCAB)],
                        tab_v.at[0, pl.ds(k * 8 * VOCAB, 8 * VOCAB)])
    pltpu.sync_copy(lse_hbm, lse_v)
    pltpu.sync_copy(idxT_hbm.at[pl.ds(0, B)], idx_v.at[0])
    pltpu.sync_copy(tgtT_hbm.at[pl.ds(0, B)], tgt_v.at[0])

    zero = jnp.zeros((L,), jnp.int32)

    def t_body(t, acc):
        q = t % 2

        @pl.when(t >= 1)
        def _():
            pltpu.make_async_copy(idxT_hbm.at[pl.ds(t * B, B)],
                                  idx_v.at[q], sem2).wait()
            pltpu.make_async_copy(tgtT_hbm.at[pl.ds(t * B, B)],
                                  tgt_v.at[q], sem2).wait()

        @pl.when(t + 1 < T)
        def _():
            pltpu.async_copy(idxT_hbm.at[pl.ds((t + 1) * B, B)],
                             idx_v.at[1 - q], sem2)
            pltpu.async_copy(tgtT_hbm.at[pl.ds((t + 1) * B, B)],
                             tgt_v.at[1 - q], sem2)

        for k in range(KPW):
            crk = jnp.minimum(cr0 + k, NCR - 1)
            par = (t * KPW + k) % 2
            drain = pltpu.make_async_copy(
                obuf_v.at[par], out_hbm.at[t, crk], sem)
            if k >= 2:
                drain.wait()
            else:
                @pl.when(t >= 1)
                def _():
                    drain.wait()

            @plsc.parallel_loop(0, NG, 1, unroll=8)
            def _(g):
                iv = idx_v[q, pl.ds(g * L, L)]
                off = (g >> 3) * 1024 + (g & 7) * L
                for s in range(8):
                    vals = plsc.load_gather(
                        tab_v, [zero, iv + (k * 8 * VOCAB + s * VOCAB)])
                    obuf_v[par, pl.ds(off + s * 128, L)] = vals

            pltpu.async_copy(obuf_v.at[par], out_hbm.at[t, crk], sem)

        @plsc.parallel_loop(0, NG, 1, unroll=4, carry=acc)
        def loss_acc(g, a):
            iv = idx_v[q, pl.ds(g * L, L)]
            tv = tgt_v[q, pl.ds(g * L, L)]
            rel = (tv >> 3) - cr0
            m = (rel >= 0) & (rel < KPW)
            relc = jnp.clip(rel, 0, KPW - 1)
            addr = relc * (8 * VOCAB) + (tv & 7) * VOCAB + iv
            picks = plsc.load_gather(tab_v, [zero, addr], mask=m)
            lsev = plsc.load_gather(lse_v, [zero, iv], mask=m)
            return a + jnp.where(m, lsev - picks, 0.0)

        return loss_acc

    acc = lax.fori_loop(0, T, t_body, jnp.zeros((L,), jnp.float32))

    for j in range(2):
        pltpu.make_async_copy(obuf_v.at[j], out_hbm.at[0, 0], sem).wait()

    acc_v[...] = acc
    pltpu.sync_copy(acc_v, part_hbm.at[wid])


def kernel(idx, targets, table):
    lse = _row_lse(table).reshape(1, VOCAB)
    tableT_flat = table.T.reshape(VOCAB * VOCAB)
    idxT_flat = idx.T.reshape(BT)
    tgtT_flat = targets.T.reshape(BT)
    out5, parts = _sc_gather_loss(tableT_flat, idxT_flat, tgtT_flat, lse)
    logits = (out5.reshape(T, NCR, B // 128, 8, 128)
              .transpose(2, 4, 0, 1, 3)
              .reshape(B, T, VOCAB))
    loss = jnp.sum(parts) / BT
    return (logits, loss)
